# baseline (speedup 1.0000x reference)
import functools

import jax
import jax.numpy as jnp
from jax import lax
from jax.experimental import pallas as pl
from jax.experimental.pallas import tpu as pltpu
from jax.experimental.pallas import tpu_sc as plsc

N_NODES = 10000
N_EDGES = 320000
D = 128
L = 16
NC, NS = 2, 16
NW = NC * NS
EPW = N_EDGES // NW
B = 80
NCHUNK = EPW // B
NG = B // L


def _sc_body(c_hbm, g_hbm, src_hbm, dst_hbm, gs_hbm, cs_hbm, w_hbm, wp_hbm,
             mu_out, xd_out, pi_out,
             sidx, didx, c_r0, g_r0, c_r1, g_r1, gs_tab, cs_tab, wbuf, wpk,
             mu_st, xd_st, pi_st,
             sem_c0, sem_g0, sem_c1, sem_g1):
    wid = lax.axis_index("s") * NC + lax.axis_index("c")
    c_rows = (c_r0, c_r1)
    g_rows = (g_r0, g_r1)
    sems_c = (sem_c0, sem_c1)
    sems_g = (sem_g0, sem_g1)

    pltpu.sync_copy(gs_hbm, gs_tab)
    pltpu.sync_copy(cs_hbm, cs_tab)
    pltpu.sync_copy(w_hbm, wbuf)
    pltpu.sync_copy(wp_hbm, wpk)
    pltpu.sync_copy(src_hbm.at[wid], sidx)
    pltpu.sync_copy(dst_hbm.at[wid], didx)
    bvec = wbuf[3, pl.ds(0, L)]
    b_mean = bvec[0]
    b_disp = bvec[1]
    b_pi = bvec[2]
    lanes = jnp.arange(L, dtype=jnp.int32)
    wlo = [[None] * (D // 32) for _ in range(3)]
    whi = [[None] * (D // 32) for _ in range(3)]
    for k in range(3):
        for c4 in range(D // 32):
            wv = plsc.bitcast(wpk[pl.ds(k * (D // 2) + c4 * L, L)],
                              jnp.bfloat16)
            lo, hi = plsc.unpack(wv, format=plsc.PackFormat.INTERLEAVED)
            wlo[k][c4] = lo
            whi[k][c4] = hi

    def start(ci, b):
        pltpu.async_copy(c_hbm.at[sidx.at[ci]], c_rows[b], sems_c[b])
        pltpu.async_copy(g_hbm.at[didx.at[ci]], g_rows[b], sems_g[b])

    def drain(b):
        dummy = c_hbm.at[pl.ds(0, B)]
        pltpu.make_async_copy(dummy, c_rows[b], sems_c[b]).wait()
        pltpu.make_async_copy(dummy, g_rows[b], sems_g[b]).wait()

    def compute(ci, b):

        def group_body(t, carry2):
            z = jnp.zeros((L,), jnp.float32)
            dm = z
            dd = z
            dp = z
            for el in range(L):
                e = t * L + el
                a0 = a1 = a2 = a3 = a4 = a5 = z
                for c4 in range(D // 32):
                    csl = pl.ds(c4 * L, L)
                    cv = plsc.bitcast(c_rows[b][e, csl], jnp.bfloat16)
                    gv = plsc.bitcast(g_rows[b][e, csl], jnp.bfloat16)
                    h = cv * gv
                    hlo, hhi = plsc.unpack(
                        h, format=plsc.PackFormat.INTERLEAVED)
                    a0 = a0 + hlo * wlo[0][c4]
                    a1 = a1 + hlo * wlo[1][c4]
                    a2 = a2 + hlo * wlo[2][c4]
                    a3 = a3 + hhi * whi[0][c4]
                    a4 = a4 + hhi * whi[1][c4]
                    a5 = a5 + hhi * whi[2][c4]
                msk = lanes == el
                dm = jnp.where(msk, jnp.sum(a0 + a3), dm)
                dd = jnp.where(msk, jnp.sum(a1 + a4), dd)
                dp = jnp.where(msk, jnp.sum(a2 + a5), dp)

            sl = pl.ds(t * L, L)
            gsv = plsc.load_gather(gs_tab, [didx[ci, sl]])
            csv = plsc.load_gather(cs_tab, [sidx[ci, sl]])
            mu_ = gsv * (1.0 / (1.0 + jnp.exp(-(dm + b_mean))))
            mu = csv * jnp.clip(jnp.exp(mu_) - 1.0, 1e-5, 1e6)
            piv = 1.0 / (1.0 + jnp.exp(-(dp + b_pi)))
            x = gsv * (dd + b_disp)
            v = 1.0 + jnp.exp(-jnp.abs(x))
            bits = plsc.bitcast(v, jnp.int32)
            ef = ((bits >> 23) - 127).astype(jnp.float32)
            tm = plsc.bitcast((bits & 0x007FFFFF) | 0x3F800000,
                              jnp.float32) - 1.0
            p = jnp.float32(1.477872077e-02)
            for coef in (-7.684872596e-02, 1.904208314e-01, -3.231159351e-01,
                         4.724995252e-01, -7.203866119e-01, 1.442652111e+00,
                         3.196978290e-07):
                p = p * tm + jnp.float32(coef)
            ln_v = (ef + p) * jnp.float32(0.6931471805599453)
            disp = jnp.clip(jnp.maximum(x, 0.0) + ln_v, 1e-4, 1e4)
            osl = pl.ds(ci * B + t * L, L)
            mu_st[osl] = mu
            xd_st[osl] = disp
            pi_st[osl] = piv
            return carry2

        lax.fori_loop(0, NG, group_body, 0)

    start(0, 0)

    def outer(k, carry):
        ci0 = 2 * k
        start(ci0 + 1, 1)
        drain(0)
        compute(ci0, 0)
        start(ci0 + 2, 0)
        drain(1)
        compute(ci0 + 1, 1)
        return carry

    lax.fori_loop(0, (NCHUNK - 1) // 2, outer, 0)
    drain(0)
    compute(NCHUNK - 1, 0)

    obase = pl.ds(wid * EPW, EPW)
    pltpu.sync_copy(mu_st, mu_out.at[obase])
    pltpu.sync_copy(xd_st, xd_out.at[obase])
    pltpu.sync_copy(pi_st, pi_out.at[obase])


@jax.jit
def _sc_call(c_feat, g_feat, src, dst, gs, cs, wb, wp):
    f32 = jnp.float32
    mesh = plsc.VectorSubcoreMesh(core_axis_name="c", subcore_axis_name="s")
    src3 = src.reshape(NW, NCHUNK, B)
    dst3 = dst.reshape(NW, NCHUNK, B)
    call = pl.kernel(
        _sc_body,
        out_type=[jax.ShapeDtypeStruct((N_EDGES,), f32)] * 3,
        mesh=mesh,
        compiler_params=pltpu.CompilerParams(
            needs_layout_passes=False, use_tc_tiling_on_sc=False),
        scratch_types=[
            pltpu.VMEM((NCHUNK, B), jnp.int32),
            pltpu.VMEM((NCHUNK, B), jnp.int32),
            pltpu.VMEM((B, D // 2), jnp.int32),
            pltpu.VMEM((B, D // 2), jnp.int32),
            pltpu.VMEM((B, D // 2), jnp.int32),
            pltpu.VMEM((B, D // 2), jnp.int32),
            pltpu.VMEM((N_NODES,), f32),
            pltpu.VMEM((N_NODES,), f32),
            pltpu.VMEM((4, D), f32),
            pltpu.VMEM((3 * D // 2,), jnp.int32),
            pltpu.VMEM((EPW,), f32),
            pltpu.VMEM((EPW,), f32),
            pltpu.VMEM((EPW,), f32),
            pltpu.SemaphoreType.DMA,
            pltpu.SemaphoreType.DMA,
            pltpu.SemaphoreType.DMA,
            pltpu.SemaphoreType.DMA,
        ],
    )
    c_pk = lax.bitcast_convert_type(
        c_feat.astype(jnp.bfloat16).reshape(N_NODES, D // 2, 2), jnp.int32)
    g_pk = lax.bitcast_convert_type(
        g_feat.astype(jnp.bfloat16).reshape(N_NODES, D // 2, 2), jnp.int32)
    return call(c_pk, g_pk, src3, dst3, gs, cs, wb, wp)


def kernel(c_feat, g_feat, edge_index, gs_factor, cs_factor,
           W_mean, b_mean, W_disp, b_disp, W_pi, b_pi):
    src = edge_index[0].astype(jnp.int32)
    dst = edge_index[1].astype(jnp.int32)
    brow = jnp.concatenate(
        [b_mean, b_disp, b_pi, jnp.zeros((D - 3,), jnp.float32)])
    wb = jnp.concatenate([W_mean, W_disp, W_pi, brow[None, :]], axis=0)
    wp = lax.bitcast_convert_type(
        jnp.concatenate([W_mean, W_disp, W_pi],
                        axis=0).astype(jnp.bfloat16).reshape(3 * D // 2, 2),
        jnp.int32)
    mu, disp, piv = _sc_call(c_feat, g_feat, src, dst,
                             gs_factor[:, 0], cs_factor[:, 0], wb, wp)
    return (mu.reshape(N_EDGES, 1), disp.reshape(N_EDGES, 1),
            piv.reshape(N_EDGES, 1))


# speedup vs baseline: 1.2550x; 1.2550x over previous
import jax
import jax.numpy as jnp
from jax import lax
from jax.experimental import pallas as pl
from jax.experimental.pallas import tpu as pltpu
from jax.experimental.pallas import tpu_sc as plsc

N_NODES = 10000
N_EDGES = 320000
D = 128
L = 16
NC, NS = 2, 16
NW = NC * NS
EPW = N_EDGES // NW
B = 80
NCHUNK = EPW // B
NG = B // L


def _sc_body(c_hbm, g_hbm, src_hbm, dst_hbm, gs_hbm, cs_hbm, w_hbm,
             mu_out, xd_out, pi_out,
             sidx, didx, c_r0, g_r0, c_r1, g_r1, gs_tab, cs_tab, wbuf,
             mu_st, xd_st, pi_st,
             sem_c0, sem_g0, sem_c1, sem_g1):
    wid = lax.axis_index("s") * NC + lax.axis_index("c")
    c_rows = (c_r0, c_r1)
    g_rows = (g_r0, g_r1)
    sems_c = (sem_c0, sem_c1)
    sems_g = (sem_g0, sem_g1)

    pltpu.sync_copy(gs_hbm, gs_tab)
    pltpu.sync_copy(cs_hbm, cs_tab)
    pltpu.sync_copy(w_hbm, wbuf)
    pltpu.sync_copy(src_hbm.at[wid], sidx)
    pltpu.sync_copy(dst_hbm.at[wid], didx)
    bvec = wbuf[3, pl.ds(0, L)]
    b_mean = bvec[0]
    b_disp = bvec[1]
    b_pi = bvec[2]
    lanes = jnp.arange(L, dtype=jnp.int32)
    w0c = [wbuf[0, pl.ds(j0 * L, L)] for j0 in range(D // L)]
    w1c = [wbuf[1, pl.ds(j0 * L, L)] for j0 in range(D // L)]
    w2c = [wbuf[2, pl.ds(j0 * L, L)] for j0 in range(D // L)]

    def start(ci, b):
        pltpu.async_copy(c_hbm.at[sidx.at[ci]], c_rows[b], sems_c[b])
        pltpu.async_copy(g_hbm.at[didx.at[ci]], g_rows[b], sems_g[b])

    def drain(b):
        dummy = c_hbm.at[pl.ds(0, B)]
        pltpu.make_async_copy(dummy, c_rows[b], sems_c[b]).wait()
        pltpu.make_async_copy(dummy, g_rows[b], sems_g[b]).wait()

    def compute(ci, b):

        def group_body(t, carry2):
            z = jnp.zeros((L,), jnp.float32)
            dm = z
            dd = z
            dp = z
            for el in range(L):
                e = t * L + el
                a0 = a1 = a2 = None
                for j0 in range(D // L):
                    jsl = pl.ds(j0 * L, L)
                    cv = c_rows[b][e, jsl]
                    gv = g_rows[b][e, jsl]
                    h = cv * gv
                    if j0 == 0:
                        a0 = h * w0c[j0]
                        a1 = h * w1c[j0]
                        a2 = h * w2c[j0]
                    else:
                        a0 = a0 + h * w0c[j0]
                        a1 = a1 + h * w1c[j0]
                        a2 = a2 + h * w2c[j0]
                msk = lanes == el
                dm = jnp.where(msk, jnp.sum(a0), dm)
                dd = jnp.where(msk, jnp.sum(a1), dd)
                dp = jnp.where(msk, jnp.sum(a2), dp)

            sl = pl.ds(t * L, L)
            gsv = plsc.load_gather(gs_tab, [didx[ci, sl]])
            csv = plsc.load_gather(cs_tab, [sidx[ci, sl]])
            mu_ = gsv * (1.0 / (1.0 + jnp.exp(-(dm + b_mean))))
            mu = csv * jnp.clip(jnp.exp(mu_) - 1.0, 1e-5, 1e6)
            piv = 1.0 / (1.0 + jnp.exp(-(dp + b_pi)))
            x = gsv * (dd + b_disp)
            v = 1.0 + jnp.exp(-jnp.abs(x))
            bits = plsc.bitcast(v, jnp.int32)
            ef = ((bits >> 23) - 127).astype(jnp.float32)
            tm = plsc.bitcast((bits & 0x007FFFFF) | 0x3F800000,
                              jnp.float32) - 1.0
            p = jnp.float32(1.477872077e-02)
            for coef in (-7.684872596e-02, 1.904208314e-01, -3.231159351e-01,
                         4.724995252e-01, -7.203866119e-01, 1.442652111e+00,
                         3.196978290e-07):
                p = p * tm + jnp.float32(coef)
            ln_v = (ef + p) * jnp.float32(0.6931471805599453)
            disp = jnp.clip(jnp.maximum(x, 0.0) + ln_v, 1e-4, 1e4)
            osl = pl.ds(ci * B + t * L, L)
            mu_st[osl] = mu
            xd_st[osl] = disp
            pi_st[osl] = piv
            return carry2

        lax.fori_loop(0, NG, group_body, 0)

    start(0, 0)

    def outer(k, carry):
        ci0 = 2 * k
        start(ci0 + 1, 1)
        drain(0)
        compute(ci0, 0)
        start(ci0 + 2, 0)
        drain(1)
        compute(ci0 + 1, 1)
        return carry

    lax.fori_loop(0, (NCHUNK - 1) // 2, outer, 0)
    drain(0)
    compute(NCHUNK - 1, 0)

    obase = pl.ds(wid * EPW, EPW)
    pltpu.sync_copy(mu_st, mu_out.at[obase])
    pltpu.sync_copy(xd_st, xd_out.at[obase])
    pltpu.sync_copy(pi_st, pi_out.at[obase])


@jax.jit
def _sc_call(c_feat, g_feat, src, dst, gs, cs, wb):
    f32 = jnp.float32
    mesh = plsc.VectorSubcoreMesh(core_axis_name="c", subcore_axis_name="s")
    src3 = src.reshape(NW, NCHUNK, B)
    dst3 = dst.reshape(NW, NCHUNK, B)
    call = pl.kernel(
        _sc_body,
        out_type=[jax.ShapeDtypeStruct((N_EDGES,), f32)] * 3,
        mesh=mesh,
        compiler_params=pltpu.CompilerParams(needs_layout_passes=False),
        scratch_types=[
            pltpu.VMEM((NCHUNK, B), jnp.int32),
            pltpu.VMEM((NCHUNK, B), jnp.int32),
            pltpu.VMEM((B, D), f32),
            pltpu.VMEM((B, D), f32),
            pltpu.VMEM((B, D), f32),
            pltpu.VMEM((B, D), f32),
            pltpu.VMEM((N_NODES,), f32),
            pltpu.VMEM((N_NODES,), f32),
            pltpu.VMEM((4, D), f32),
            pltpu.VMEM((EPW,), f32),
            pltpu.VMEM((EPW,), f32),
            pltpu.VMEM((EPW,), f32),
            pltpu.SemaphoreType.DMA,
            pltpu.SemaphoreType.DMA,
            pltpu.SemaphoreType.DMA,
            pltpu.SemaphoreType.DMA,
        ],
    )
    return call(c_feat, g_feat, src3, dst3, gs, cs, wb)


def kernel(c_feat, g_feat, edge_index, gs_factor, cs_factor,
           W_mean, b_mean, W_disp, b_disp, W_pi, b_pi):
    src = edge_index[0].astype(jnp.int32)
    dst = edge_index[1].astype(jnp.int32)
    brow = jnp.concatenate(
        [b_mean, b_disp, b_pi, jnp.zeros((D - 3,), jnp.float32)])
    wb = jnp.concatenate([W_mean, W_disp, W_pi, brow[None, :]], axis=0)
    mu, disp, piv = _sc_call(c_feat, g_feat, src, dst,
                             gs_factor[:, 0], cs_factor[:, 0], wb)
    return (mu.reshape(N_EDGES, 1), disp.reshape(N_EDGES, 1),
            piv.reshape(N_EDGES, 1))


# speedup vs baseline: 1.3187x; 1.0508x over previous
import jax
import jax.numpy as jnp
from jax import lax
from jax.experimental import pallas as pl
from jax.experimental.pallas import tpu as pltpu
from jax.experimental.pallas import tpu_sc as plsc

N_NODES = 10000
N_EDGES = 320000
D = 128
L = 16
NC, NS = 2, 16
NW = NC * NS
EPW = N_EDGES // NW
B = 80
NCHUNK = EPW // B
NG = B // L


def _sc_body(c_hbm, g_hbm, src_hbm, dst_hbm, gs_hbm, cs_hbm, w_hbm,
             mu_out, xd_out, pi_out,
             sidx, didx, c_r0, g_r0, c_r1, g_r1, gs_tab, cs_tab, wbuf,
             mu_st, xd_st, pi_st,
             sem_c0, sem_g0, sem_c1, sem_g1):
    wid = lax.axis_index("s") * NC + lax.axis_index("c")
    c_rows = (c_r0, c_r1)
    g_rows = (g_r0, g_r1)
    sems_c = (sem_c0, sem_c1)
    sems_g = (sem_g0, sem_g1)

    pltpu.sync_copy(gs_hbm, gs_tab)
    pltpu.sync_copy(cs_hbm, cs_tab)
    pltpu.sync_copy(w_hbm, wbuf)
    pltpu.sync_copy(src_hbm.at[wid], sidx)
    pltpu.sync_copy(dst_hbm.at[wid], didx)
    bvec = wbuf[3, pl.ds(0, L)]
    b_mean = bvec[0]
    b_disp = bvec[1]
    b_pi = bvec[2]
    lanes = jnp.arange(L, dtype=jnp.int32)

    def start(ci, b):
        pltpu.async_copy(c_hbm.at[sidx.at[ci]], c_rows[b], sems_c[b])
        pltpu.async_copy(g_hbm.at[didx.at[ci]], g_rows[b], sems_g[b])

    def drain(b):
        dummy = c_hbm.at[pl.ds(0, B)]
        pltpu.make_async_copy(dummy, c_rows[b], sems_c[b]).wait()
        pltpu.make_async_copy(dummy, g_rows[b], sems_g[b]).wait()

    def compute(ci, b):

        def group_body(t, carry2):
            w0c = [wbuf[0, pl.ds(j0 * L, L)] for j0 in range(D // L)]
            w1c = [wbuf[1, pl.ds(j0 * L, L)] for j0 in range(D // L)]
            w2c = [wbuf[2, pl.ds(j0 * L, L)] for j0 in range(D // L)]
            z = jnp.zeros((L,), jnp.float32)
            dm = z
            dd = z
            dp = z
            for el in range(L):
                e = t * L + el
                a0 = a1 = a2 = None
                for j0 in range(D // L):
                    jsl = pl.ds(j0 * L, L)
                    cv = c_rows[b][e, jsl]
                    gv = g_rows[b][e, jsl]
                    h = cv * gv
                    if j0 == 0:
                        a0 = h * w0c[j0]
                        a1 = h * w1c[j0]
                        a2 = h * w2c[j0]
                    else:
                        a0 = a0 + h * w0c[j0]
                        a1 = a1 + h * w1c[j0]
                        a2 = a2 + h * w2c[j0]
                msk = lanes == el
                dm = jnp.where(msk, jnp.sum(a0), dm)
                dd = jnp.where(msk, jnp.sum(a1), dd)
                dp = jnp.where(msk, jnp.sum(a2), dp)

            sl = pl.ds(t * L, L)
            gsv = plsc.load_gather(gs_tab, [didx[ci, sl]])
            csv = plsc.load_gather(cs_tab, [sidx[ci, sl]])
            mu_ = gsv * (1.0 / (1.0 + jnp.exp(-(dm + b_mean))))
            mu = csv * jnp.clip(jnp.exp(mu_) - 1.0, 1e-5, 1e6)
            piv = 1.0 / (1.0 + jnp.exp(-(dp + b_pi)))
            x = gsv * (dd + b_disp)
            v = 1.0 + jnp.exp(-jnp.abs(x))
            bits = plsc.bitcast(v, jnp.int32)
            ef = ((bits >> 23) - 127).astype(jnp.float32)
            tm = plsc.bitcast((bits & 0x007FFFFF) | 0x3F800000,
                              jnp.float32) - 1.0
            p = jnp.float32(1.477872077e-02)
            for coef in (-7.684872596e-02, 1.904208314e-01, -3.231159351e-01,
                         4.724995252e-01, -7.203866119e-01, 1.442652111e+00,
                         3.196978290e-07):
                p = p * tm + jnp.float32(coef)
            ln_v = (ef + p) * jnp.float32(0.6931471805599453)
            disp = jnp.clip(jnp.maximum(x, 0.0) + ln_v, 1e-4, 1e4)
            osl = pl.ds(ci * B + t * L, L)
            mu_st[osl] = mu
            xd_st[osl] = disp
            pi_st[osl] = piv
            return carry2

        lax.fori_loop(0, NG, group_body, 0)

    start(0, 0)

    def outer(k, carry):
        ci0 = 2 * k
        start(ci0 + 1, 1)
        drain(0)
        compute(ci0, 0)
        start(ci0 + 2, 0)
        drain(1)
        compute(ci0 + 1, 1)
        return carry

    lax.fori_loop(0, (NCHUNK - 1) // 2, outer, 0)
    drain(0)
    compute(NCHUNK - 1, 0)

    obase = pl.ds(wid * EPW, EPW)
    pltpu.sync_copy(mu_st, mu_out.at[obase])
    pltpu.sync_copy(xd_st, xd_out.at[obase])
    pltpu.sync_copy(pi_st, pi_out.at[obase])


@jax.jit
def _sc_call(c_feat, g_feat, src, dst, gs, cs, wb):
    f32 = jnp.float32
    mesh = plsc.VectorSubcoreMesh(core_axis_name="c", subcore_axis_name="s")
    src3 = src.reshape(NW, NCHUNK, B)
    dst3 = dst.reshape(NW, NCHUNK, B)
    call = pl.kernel(
        _sc_body,
        out_type=[jax.ShapeDtypeStruct((N_EDGES,), f32)] * 3,
        mesh=mesh,
        compiler_params=pltpu.CompilerParams(needs_layout_passes=False),
        scratch_types=[
            pltpu.VMEM((NCHUNK, B), jnp.int32),
            pltpu.VMEM((NCHUNK, B), jnp.int32),
            pltpu.VMEM((B, D), f32),
            pltpu.VMEM((B, D), f32),
            pltpu.VMEM((B, D), f32),
            pltpu.VMEM((B, D), f32),
            pltpu.VMEM((N_NODES,), f32),
            pltpu.VMEM((N_NODES,), f32),
            pltpu.VMEM((4, D), f32),
            pltpu.VMEM((EPW,), f32),
            pltpu.VMEM((EPW,), f32),
            pltpu.VMEM((EPW,), f32),
            pltpu.SemaphoreType.DMA,
            pltpu.SemaphoreType.DMA,
            pltpu.SemaphoreType.DMA,
            pltpu.SemaphoreType.DMA,
        ],
    )
    return call(c_feat, g_feat, src3, dst3, gs, cs, wb)


def kernel(c_feat, g_feat, edge_index, gs_factor, cs_factor,
           W_mean, b_mean, W_disp, b_disp, W_pi, b_pi):
    src = edge_index[0].astype(jnp.int32)
    dst = edge_index[1].astype(jnp.int32)
    brow = jnp.concatenate(
        [b_mean, b_disp, b_pi, jnp.zeros((D - 3,), jnp.float32)])
    wb = jnp.concatenate([W_mean, W_disp, W_pi, brow[None, :]], axis=0)
    mu, disp, piv = _sc_call(c_feat, g_feat, src, dst,
                             gs_factor[:, 0], cs_factor[:, 0], wb)
    return (mu.reshape(N_EDGES, 1), disp.reshape(N_EDGES, 1),
            piv.reshape(N_EDGES, 1))


# speedup vs baseline: 1.3318x; 1.0099x over previous
import jax
import jax.numpy as jnp
from jax import lax
from jax.experimental import pallas as pl
from jax.experimental.pallas import tpu as pltpu
from jax.experimental.pallas import tpu_sc as plsc

N_NODES = 10000
N_EDGES = 320000
D = 128
L = 16
NC, NS = 2, 16
NW = NC * NS
EPW = N_EDGES // NW
B = 80
NCHUNK = EPW // B
NG = B // L


def _sc_body(c_hbm, g_hbm, src_hbm, dst_hbm, gs_hbm, cs_hbm, w_hbm,
             mu_out, xd_out, pi_out,
             sidx, didx, c_r0, g_r0, c_r1, g_r1, gs_tab, cs_tab, wbuf,
             mu_st, xd_st, pi_st,
             sem_c0, sem_g0, sem_c1, sem_g1,
             sem_a, sem_b, sem_d, sem_e, sem_f):
    wid = lax.axis_index("s") * NC + lax.axis_index("c")
    c_rows = (c_r0, c_r1)
    g_rows = (g_r0, g_r1)
    sems_c = (sem_c0, sem_c1)
    sems_g = (sem_g0, sem_g1)

    cp_i = pltpu.async_copy(src_hbm.at[wid], sidx, sem_a)
    cp_j = pltpu.async_copy(dst_hbm.at[wid], didx, sem_b)
    cp_gs = pltpu.async_copy(gs_hbm, gs_tab, sem_d)
    cp_cs = pltpu.async_copy(cs_hbm, cs_tab, sem_e)
    cp_w = pltpu.async_copy(w_hbm, wbuf, sem_f)
    cp_i.wait()
    cp_j.wait()
    cp_gs.wait()
    cp_cs.wait()
    cp_w.wait()
    bvec = wbuf[3, pl.ds(0, L)]
    b_mean = bvec[0]
    b_disp = bvec[1]
    b_pi = bvec[2]
    lanes = jnp.arange(L, dtype=jnp.int32)

    def start(ci, b):
        pltpu.async_copy(c_hbm.at[sidx.at[ci]], c_rows[b], sems_c[b])
        pltpu.async_copy(g_hbm.at[didx.at[ci]], g_rows[b], sems_g[b])

    def drain(b):
        dummy = c_hbm.at[pl.ds(0, B)]
        pltpu.make_async_copy(dummy, c_rows[b], sems_c[b]).wait()
        pltpu.make_async_copy(dummy, g_rows[b], sems_g[b]).wait()

    def compute(ci, b):

        def group_body(t, carry2):
            w0c = [wbuf[0, pl.ds(j0 * L, L)] for j0 in range(D // L)]
            w1c = [wbuf[1, pl.ds(j0 * L, L)] for j0 in range(D // L)]
            w2c = [wbuf[2, pl.ds(j0 * L, L)] for j0 in range(D // L)]
            base = ci * B + t * L
            z = jnp.zeros((L,), jnp.float32)
            dm = z
            dd = z
            dp = z
            for el in range(L):
                e = t * L + el
                a0 = a1 = a2 = None
                for j0 in range(D // L):
                    jsl = pl.ds(j0 * L, L)
                    cv = c_rows[b][e, jsl]
                    gv = g_rows[b][e, jsl]
                    h = cv * gv
                    if j0 == 0:
                        a0 = h * w0c[j0]
                        a1 = h * w1c[j0]
                        a2 = h * w2c[j0]
                    else:
                        a0 = a0 + h * w0c[j0]
                        a1 = a1 + h * w1c[j0]
                        a2 = a2 + h * w2c[j0]
                msk = lanes == el
                dm = jnp.where(msk, jnp.sum(a0), dm)
                dd = jnp.where(msk, jnp.sum(a1), dd)
                dp = jnp.where(msk, jnp.sum(a2), dp)

            sl = pl.ds(t * L, L)
            osl = pl.ds(base, L)
            gsv = plsc.load_gather(gs_tab, [didx[ci, sl]])
            csv = plsc.load_gather(cs_tab, [sidx[ci, sl]])
            mu_ = gsv * (1.0 / (1.0 + jnp.exp(-(dm + b_mean))))
            mu = csv * jnp.clip(jnp.exp(mu_) - 1.0, 1e-5, 1e6)
            piv = 1.0 / (1.0 + jnp.exp(-(dp + b_pi)))
            x = gsv * (dd + b_disp)
            v = 1.0 + jnp.exp(-jnp.abs(x))
            bits = plsc.bitcast(v, jnp.int32)
            ef = ((bits >> 23) - 127).astype(jnp.float32)
            tm = plsc.bitcast((bits & 0x007FFFFF) | 0x3F800000,
                              jnp.float32) - 1.0
            p = jnp.float32(1.477872077e-02)
            for coef in (-7.684872596e-02, 1.904208314e-01, -3.231159351e-01,
                         4.724995252e-01, -7.203866119e-01, 1.442652111e+00,
                         3.196978290e-07):
                p = p * tm + jnp.float32(coef)
            ln_v = (ef + p) * jnp.float32(0.6931471805599453)
            disp = jnp.clip(jnp.maximum(x, 0.0) + ln_v, 1e-4, 1e4)
            mu_st[osl] = mu
            xd_st[osl] = disp
            pi_st[osl] = piv
            return carry2

        lax.fori_loop(0, NG, group_body, 0)

    start(0, 0)

    def outer(k, carry):
        ci0 = 2 * k
        start(ci0 + 1, 1)
        drain(0)
        compute(ci0, 0)
        start(ci0 + 2, 0)
        drain(1)
        compute(ci0 + 1, 1)
        return carry

    lax.fori_loop(0, (NCHUNK - 1) // 2, outer, 0)
    drain(0)
    compute(NCHUNK - 1, 0)

    obase = pl.ds(wid * EPW, EPW)
    lsl = pl.ds(0, EPW)
    cp1 = pltpu.async_copy(mu_st.at[lsl], mu_out.at[obase], sem_a)
    cp2 = pltpu.async_copy(xd_st.at[lsl], xd_out.at[obase], sem_b)
    cp3 = pltpu.async_copy(pi_st.at[lsl], pi_out.at[obase], sem_d)
    cp1.wait()
    cp2.wait()
    cp3.wait()


@jax.jit
def _sc_call(c_feat, g_feat, src, dst, gs, cs, wb):
    f32 = jnp.float32
    mesh = plsc.VectorSubcoreMesh(core_axis_name="c", subcore_axis_name="s")
    src3 = src.reshape(NW, NCHUNK, B)
    dst3 = dst.reshape(NW, NCHUNK, B)
    call = pl.kernel(
        _sc_body,
        out_type=[jax.ShapeDtypeStruct((N_EDGES,), f32)] * 3,
        mesh=mesh,
        compiler_params=pltpu.CompilerParams(needs_layout_passes=False),
        scratch_types=[
            pltpu.VMEM((NCHUNK, B), jnp.int32),
            pltpu.VMEM((NCHUNK, B), jnp.int32),
            pltpu.VMEM((B, D), f32),
            pltpu.VMEM((B, D), f32),
            pltpu.VMEM((B, D), f32),
            pltpu.VMEM((B, D), f32),
            pltpu.VMEM((N_NODES,), f32),
            pltpu.VMEM((N_NODES,), f32),
            pltpu.VMEM((4, D), f32),
            pltpu.VMEM((EPW + L,), f32),
            pltpu.VMEM((EPW + L,), f32),
            pltpu.VMEM((EPW + L,), f32),
            pltpu.SemaphoreType.DMA,
            pltpu.SemaphoreType.DMA,
            pltpu.SemaphoreType.DMA,
            pltpu.SemaphoreType.DMA,
            pltpu.SemaphoreType.DMA,
            pltpu.SemaphoreType.DMA,
            pltpu.SemaphoreType.DMA,
            pltpu.SemaphoreType.DMA,
            pltpu.SemaphoreType.DMA,
        ],
    )
    return call(c_feat, g_feat, src3, dst3, gs, cs, wb)


def kernel(c_feat, g_feat, edge_index, gs_factor, cs_factor,
           W_mean, b_mean, W_disp, b_disp, W_pi, b_pi):
    src = edge_index[0].astype(jnp.int32)
    dst = edge_index[1].astype(jnp.int32)
    brow = jnp.concatenate(
        [b_mean, b_disp, b_pi, jnp.zeros((D - 3,), jnp.float32)])
    wb = jnp.concatenate([W_mean, W_disp, W_pi, brow[None, :]], axis=0)
    mu, disp, piv = _sc_call(c_feat, g_feat, src, dst,
                             gs_factor[:, 0], cs_factor[:, 0], wb)
    return (mu.reshape(N_EDGES, 1), disp.reshape(N_EDGES, 1),
            piv.reshape(N_EDGES, 1))
